# baseline (speedup 1.0000x reference)
import jax
import jax.numpy as jnp
from jax import lax
from jax.experimental import pallas as pl
from jax.experimental.pallas import tpu as pltpu
from jax.experimental.pallas import tpu_sc as plsc

NUM_LABEL = 64
LANES = 16


def _sc_proto_body(xm_hbm, ys_hbm, zs_hbm, zc_hbm, out_hbm,
                   ys_v, chunk_v, acc_v, cnt_v):
    c = lax.axis_index("c")
    s = lax.axis_index("s")
    t = c * 8 + s

    @pl.when(s < 8)
    def _work():
        pltpu.sync_copy(zs_hbm, acc_v)
        pltpu.sync_copy(zc_hbm, cnt_v)
        pltpu.sync_copy(ys_hbm.at[t], ys_v)
        ones16 = jnp.ones((LANES,), jnp.float32)

        def _block(kk, _):
            pltpu.sync_copy(xm_hbm.at[t, pl.ds(kk * 64, 64)], chunk_v)

            def _group(g, _):
                yv = ys_v[pl.ds(kk * 64 + g * LANES, LANES)]
                for l in range(LANES):
                    y = yv[l]
                    r = g * LANES + l
                    plsc.addupdate(cnt_v.at[y], ones16)

                    def _chunk(j, _, y=y, r=r):
                        sl = pl.ds(j * LANES, LANES)
                        plsc.addupdate(acc_v.at[y, sl], chunk_v[r, sl])
                        return 0

                    lax.fori_loop(0, 512 // LANES, _chunk, 0)
                return 0

            lax.fori_loop(0, 4, _group, 0)
            return 0

        lax.fori_loop(0, 16, _block, 0)

        for i in range(NUM_LABEL):
            recip = 1.0 / jnp.maximum(cnt_v[i], 1.0)

            def _div(j, _, i=i, recip=recip):
                sl = pl.ds(j * LANES, LANES)
                acc_v[i, sl] = acc_v[i, sl] * recip
                return 0

            lax.fori_loop(0, 512 // LANES, _div, 0)

        pltpu.sync_copy(acc_v, out_hbm.at[pl.ds(t * NUM_LABEL, NUM_LABEL)])


def _tc_dist_body(pr_ref, mask_ref, xt_ref, out_ref):
    p = pr_ref[0]
    d = p.shape[1]
    pn = jnp.sum(p * p, axis=1)
    pm2 = (-2.0 * p).astype(jnp.bfloat16)

    x = xt_ref[0]
    xpT = jax.lax.dot_general(
        pm2, x.astype(jnp.bfloat16), (((1,), (1,)), ((), ())),
        preferred_element_type=jnp.float32)
    xn8 = jax.lax.dot_general(
        jnp.ones((8, d), jnp.float32), x * x,
        (((1,), (1,)), ((), ())),
        preferred_element_type=jnp.float32)
    d2 = jnp.maximum(xpT + xn8[0:1, :] + pn[:, None], 0.0)
    e = jnp.exp(-jnp.sqrt(d2))
    s = jnp.sum(e, axis=0, keepdims=True)
    mT = jnp.transpose(mask_ref[...], (1, 0))
    out_ref[...] = e * (1.0 / s) * mT


def kernel(xs_targ, xs_meta, ys_meta, max_N_label):
    B, N_targ, d = xs_targ.shape
    N_meta = xs_meta.shape[1]

    mesh = plsc.VectorSubcoreMesh(core_axis_name="c", subcore_axis_name="s")
    sc_protos = pl.kernel(
        _sc_proto_body,
        out_type=jax.ShapeDtypeStruct((B * NUM_LABEL, d), jnp.float32),
        mesh=mesh,
        scratch_types=[
            pltpu.VMEM((N_meta,), jnp.int32),
            pltpu.VMEM((64, d), jnp.float32),
            pltpu.VMEM((NUM_LABEL, d), jnp.float32),
            pltpu.VMEM((NUM_LABEL, LANES), jnp.float32),
        ],
    )
    protos = sc_protos(
        xs_meta, ys_meta.astype(jnp.int32),
        jnp.zeros((NUM_LABEL, d), jnp.float32),
        jnp.zeros((NUM_LABEL, LANES), jnp.float32),
    ).reshape(B, NUM_LABEL, d)

    label_mask = (jnp.arange(NUM_LABEL) < max_N_label).astype(
        jnp.float32).reshape(1, NUM_LABEL)
    out = pl.pallas_call(
        _tc_dist_body,
        grid=(B,),
        in_specs=[
            pl.BlockSpec((1, NUM_LABEL, d), lambda b: (b, 0, 0)),
            pl.BlockSpec((1, NUM_LABEL), lambda b: (0, 0)),
            pl.BlockSpec((1, N_targ, d), lambda b: (b, 0, 0)),
        ],
        out_specs=pl.BlockSpec((NUM_LABEL, N_targ), lambda b: (0, b)),
        out_shape=jax.ShapeDtypeStruct((NUM_LABEL, B * N_targ), jnp.float32),
        compiler_params=pltpu.CompilerParams(
            dimension_semantics=("arbitrary",)),
    )(protos, label_mask, xs_targ)
    return out.T


# speedup vs baseline: 1.0522x; 1.0522x over previous
import jax
import jax.numpy as jnp
from jax import lax
from jax.experimental import pallas as pl
from jax.experimental.pallas import tpu as pltpu
from jax.experimental.pallas import tpu_sc as plsc

NUM_LABEL = 64
LANES = 16


def _sc_proto_body(xm_hbm, ys_hbm, zs_hbm, zc_hbm, out_hbm,
                   ys_v, chunk_v, acc_v, cnt_v):
    c = lax.axis_index("c")
    s = lax.axis_index("s")
    t = c * 8 + s

    @pl.when(s < 8)
    def _work():
        pltpu.sync_copy(zs_hbm, acc_v)
        pltpu.sync_copy(zc_hbm, cnt_v)
        pltpu.sync_copy(ys_hbm.at[t], ys_v)
        ones16 = jnp.ones((LANES,), jnp.float32)

        def _block(kk, _):
            pltpu.sync_copy(xm_hbm.at[t, pl.ds(kk * 64, 64)], chunk_v)

            def _group(g, _):
                yv = ys_v[pl.ds(kk * 64 + g * LANES, LANES)]
                for l in range(LANES):
                    y = yv[l]
                    r = g * LANES + l
                    plsc.addupdate(cnt_v.at[y], ones16)
                    for j in range(512 // LANES):
                        sl = pl.ds(j * LANES, LANES)
                        plsc.addupdate(acc_v.at[y, sl], chunk_v[r, sl])
                return 0

            lax.fori_loop(0, 4, _group, 0)
            return 0

        lax.fori_loop(0, 16, _block, 0)

        for i in range(NUM_LABEL):
            recip = 1.0 / jnp.maximum(cnt_v[i], 1.0)
            for j in range(512 // LANES):
                sl = pl.ds(j * LANES, LANES)
                acc_v[i, sl] = acc_v[i, sl] * recip

        pltpu.sync_copy(acc_v, out_hbm.at[pl.ds(t * NUM_LABEL, NUM_LABEL)])


def _tc_dist_body(pr_ref, mask_ref, xt_ref, out_ref):
    p = pr_ref[0]
    d = p.shape[1]
    pn = jnp.sum(p * p, axis=1)
    pm2 = (-2.0 * p).astype(jnp.bfloat16)

    x = xt_ref[0]
    xpT = jax.lax.dot_general(
        pm2, x.astype(jnp.bfloat16), (((1,), (1,)), ((), ())),
        preferred_element_type=jnp.float32)
    xn8 = jax.lax.dot_general(
        jnp.ones((8, d), jnp.float32), x * x,
        (((1,), (1,)), ((), ())),
        preferred_element_type=jnp.float32)
    d2 = jnp.maximum(xpT + xn8[0:1, :] + pn[:, None], 0.0)
    e = jnp.exp(-jnp.sqrt(d2))
    s = jnp.sum(e, axis=0, keepdims=True)
    mT = jnp.transpose(mask_ref[...], (1, 0))
    out_ref[...] = e * (1.0 / s) * mT


def kernel(xs_targ, xs_meta, ys_meta, max_N_label):
    B, N_targ, d = xs_targ.shape
    N_meta = xs_meta.shape[1]

    mesh = plsc.VectorSubcoreMesh(core_axis_name="c", subcore_axis_name="s")
    sc_protos = pl.kernel(
        _sc_proto_body,
        out_type=jax.ShapeDtypeStruct((B * NUM_LABEL, d), jnp.float32),
        mesh=mesh,
        scratch_types=[
            pltpu.VMEM((N_meta,), jnp.int32),
            pltpu.VMEM((64, d), jnp.float32),
            pltpu.VMEM((NUM_LABEL, d), jnp.float32),
            pltpu.VMEM((NUM_LABEL, LANES), jnp.float32),
        ],
    )
    protos = sc_protos(
        xs_meta, ys_meta.astype(jnp.int32),
        jnp.zeros((NUM_LABEL, d), jnp.float32),
        jnp.zeros((NUM_LABEL, LANES), jnp.float32),
    ).reshape(B, NUM_LABEL, d)

    label_mask = (jnp.arange(NUM_LABEL) < max_N_label).astype(
        jnp.float32).reshape(1, NUM_LABEL)
    out = pl.pallas_call(
        _tc_dist_body,
        grid=(B,),
        in_specs=[
            pl.BlockSpec((1, NUM_LABEL, d), lambda b: (b, 0, 0)),
            pl.BlockSpec((1, NUM_LABEL), lambda b: (0, 0)),
            pl.BlockSpec((1, N_targ, d), lambda b: (b, 0, 0)),
        ],
        out_specs=pl.BlockSpec((NUM_LABEL, N_targ), lambda b: (0, b)),
        out_shape=jax.ShapeDtypeStruct((NUM_LABEL, B * N_targ), jnp.float32),
        compiler_params=pltpu.CompilerParams(
            dimension_semantics=("arbitrary",)),
    )(protos, label_mask, xs_targ)
    return out.T


# speedup vs baseline: 1.4001x; 1.3306x over previous
import jax
import jax.numpy as jnp
from jax import lax
from jax.experimental import pallas as pl
from jax.experimental.pallas import tpu as pltpu
from jax.experimental.pallas import tpu_sc as plsc

NUM_LABEL = 64
LANES = 16


def _sc_proto_body(xm_hbm, ys_hbm, zs_hbm, zc_hbm, out_hbm,
                   ys_v, chunk_v, acc_v, acc2_v, cnt_v, cnt2_v):
    c = lax.axis_index("c")
    s = lax.axis_index("s")
    t = c * 8 + s // 2
    half = s % 2
    col0 = half * 256

    pltpu.sync_copy(zs_hbm, acc_v)
    pltpu.sync_copy(zs_hbm, acc2_v)
    pltpu.sync_copy(zc_hbm, cnt_v)
    pltpu.sync_copy(zc_hbm, cnt2_v)
    pltpu.sync_copy(ys_hbm.at[t], ys_v)
    ones16 = jnp.ones((LANES,), jnp.float32)
    banks = (acc_v, acc2_v)
    cbanks = (cnt_v, cnt2_v)

    def _block(kk, _):
        pltpu.sync_copy(
            xm_hbm.at[t, pl.ds(kk * 64, 64), pl.ds(col0, 256)], chunk_v)

        def _group(g, _):
            yv = ys_v[pl.ds(kk * 64 + g * LANES, LANES)]
            for l in range(LANES):
                y = yv[l]
                r = g * LANES + l
                bank = banks[l % 2]
                plsc.addupdate(cbanks[l % 2].at[y], ones16)
                for j in range(256 // LANES):
                    sl = pl.ds(j * LANES, LANES)
                    plsc.addupdate(bank.at[y, sl], chunk_v[r, sl])
            return 0

        lax.fori_loop(0, 4, _group, 0)
        return 0

    lax.fori_loop(0, 16, _block, 0)

    for i in range(NUM_LABEL):
        recip = 1.0 / jnp.maximum(cnt_v[i] + cnt2_v[i], 1.0)
        for j in range(256 // LANES):
            sl = pl.ds(j * LANES, LANES)
            acc_v[i, sl] = (acc_v[i, sl] + acc2_v[i, sl]) * recip

    pltpu.sync_copy(
        acc_v,
        out_hbm.at[pl.ds(t * NUM_LABEL, NUM_LABEL), pl.ds(col0, 256)])


def _tc_dist_body(pr_ref, mask_ref, xt_ref, out_ref):
    p = pr_ref[0]
    d = p.shape[1]
    pn = jnp.sum(p * p, axis=1)
    pm2 = (-2.0 * p).astype(jnp.bfloat16)

    x = xt_ref[0]
    xpT = jax.lax.dot_general(
        pm2, x.astype(jnp.bfloat16), (((1,), (1,)), ((), ())),
        preferred_element_type=jnp.float32)
    xn8 = jax.lax.dot_general(
        jnp.ones((8, d), jnp.float32), x * x,
        (((1,), (1,)), ((), ())),
        preferred_element_type=jnp.float32)
    d2 = jnp.maximum(xpT + xn8[0:1, :] + pn[:, None], 0.0)
    e = jnp.exp(-jnp.sqrt(d2))
    s = jnp.sum(e, axis=0, keepdims=True)
    mT = jnp.transpose(mask_ref[...], (1, 0))
    out_ref[...] = e * (1.0 / s) * mT


def kernel(xs_targ, xs_meta, ys_meta, max_N_label):
    B, N_targ, d = xs_targ.shape
    N_meta = xs_meta.shape[1]

    mesh = plsc.VectorSubcoreMesh(core_axis_name="c", subcore_axis_name="s")
    sc_protos = pl.kernel(
        _sc_proto_body,
        out_type=jax.ShapeDtypeStruct((B * NUM_LABEL, d), jnp.float32),
        mesh=mesh,
        scratch_types=[
            pltpu.VMEM((N_meta,), jnp.int32),
            pltpu.VMEM((64, d // 2), jnp.float32),
            pltpu.VMEM((NUM_LABEL, d // 2), jnp.float32),
            pltpu.VMEM((NUM_LABEL, d // 2), jnp.float32),
            pltpu.VMEM((NUM_LABEL, LANES), jnp.float32),
            pltpu.VMEM((NUM_LABEL, LANES), jnp.float32),
        ],
    )
    protos = sc_protos(
        xs_meta, ys_meta.astype(jnp.int32),
        jnp.zeros((NUM_LABEL, d // 2), jnp.float32),
        jnp.zeros((NUM_LABEL, LANES), jnp.float32),
    ).reshape(B, NUM_LABEL, d)

    label_mask = (jnp.arange(NUM_LABEL) < max_N_label).astype(
        jnp.float32).reshape(1, NUM_LABEL)
    out = pl.pallas_call(
        _tc_dist_body,
        grid=(B,),
        in_specs=[
            pl.BlockSpec((1, NUM_LABEL, d), lambda b: (b, 0, 0)),
            pl.BlockSpec((1, NUM_LABEL), lambda b: (0, 0)),
            pl.BlockSpec((1, N_targ, d), lambda b: (b, 0, 0)),
        ],
        out_specs=pl.BlockSpec((NUM_LABEL, N_targ), lambda b: (0, b)),
        out_shape=jax.ShapeDtypeStruct((NUM_LABEL, B * N_targ), jnp.float32),
        compiler_params=pltpu.CompilerParams(
            dimension_semantics=("arbitrary",)),
    )(protos, label_mask, xs_targ)
    return out.T
